# Initial kernel scaffold; baseline (speedup 1.0000x reference)
import jax
import jax.numpy as jnp
from jax.experimental import pallas as pl


def kernel(x, edge_index, face_feats, W_e1, b_e1, bn_gamma, bn_beta, prelu_a, W_e2, b_e2, W0, b0, Wg, bg, Wface, Wf, bf):
    raise NotImplementedError("write your pallas kernel here")


# speedup vs baseline: 81.5371x; 81.5371x over previous
import functools

import jax
import jax.numpy as jnp
from jax import lax
from jax.experimental import pallas as pl
from jax.experimental.pallas import tpu as pltpu
from jax.experimental.pallas import tpu_sc as plsc

NC = 2
NS = 16
NW = NC * NS
LANE = 128
CROWS = 16


def _stats_body(xt_ref, o_ref):
    x0 = xt_ref[0:1, :]
    x1 = xt_ref[1:2, :]
    o_ref[0] = jnp.sum(x0)
    o_ref[1] = jnp.sum(x1)
    o_ref[2] = jnp.sum(x0 * x0)
    o_ref[3] = jnp.sum(x0 * x1)
    o_ref[4] = jnp.sum(x1 * x1)


def _main_body(n_rows, stats_ref, pa_ref, x_ref, face_ref, we1_ref, be1_ref,
               gam_ref, bet_ref, we2_ref, be2_ref, w0_ref, wg_ref, wface_ref,
               wf_ref, s_out, lin_out):
    ninv = 1.0 / float(n_rows)
    s0 = stats_ref[0] * ninv
    s1 = stats_ref[1] * ninv
    c00 = stats_ref[2] * ninv - s0 * s0
    c01 = stats_ref[3] * ninv - s0 * s1
    c11 = stats_ref[4] * ninv - s1 * s1
    w0r = we1_ref[0:1, :]
    w1r = we1_ref[1:2, :]
    mu = s0 * w0r + s1 * w1r + be1_ref[...]
    var = w0r * w0r * c00 + 2.0 * w0r * w1r * c01 + w1r * w1r * c11
    inv = lax.rsqrt(var + 1e-5)
    h = jnp.dot(x_ref[...], we1_ref[...],
                preferred_element_type=jnp.float32) + be1_ref[...]
    hn = (h - mu) * (inv * gam_ref[...]) + bet_ref[...]
    a = pa_ref[0]
    hp = jnp.where(hn >= 0, hn, a * hn)
    e = jnp.dot(hp, we2_ref[...],
                preferred_element_type=jnp.float32) + be2_ref[...]
    lin_out[...] = jnp.dot(e, w0_ref[...], preferred_element_type=jnp.float32)
    m = jnp.dot(e, wg_ref[...], preferred_element_type=jnp.float32)
    m = m + jnp.dot(face_ref[...], wface_ref[...],
                    preferred_element_type=jnp.float32)
    s_out[...] = jnp.dot(m, wf_ref[...], preferred_element_type=jnp.float32)


def _post_body(hist_ref, s_ref, dinv_ref, t_ref):
    deg = hist_ref[0] + hist_ref[1] + 1.0
    dinv = lax.rsqrt(deg)
    dinv_ref[...] = dinv
    t_ref[...] = s_ref[...] * dinv


def _final_body(lin_ref, s_ref, dinv_ref, acc_ref, bg_ref, wft_ref, bb_ref,
                out_ref):
    const = jnp.sum(bg_ref[...] * wft_ref[...]) + bb_ref[0] + bb_ref[1]
    dinv = dinv_ref[...]
    out_ref[...] = (lin_ref[...] + (acc_ref[0] + acc_ref[1]) * dinv
                    + s_ref[...] * dinv * dinv + const)


def _make_hist_sc(np_pad, rows_per_w, n_chunks):
    mesh = plsc.VectorSubcoreMesh(core_axis_name="c", subcore_axis_name="s")

    @functools.partial(
        pl.kernel,
        out_type=jax.ShapeDtypeStruct((NC, np_pad), jnp.float32),
        mesh=mesh,
        scratch_types=[
            pltpu.VMEM((CROWS, LANE), jnp.int32),
            pltpu.VMEM((LANE,), jnp.float32),
            pltpu.VMEM_SHARED((np_pad,), jnp.float32),
        ],
    )
    def hist_sc(dst2d, zeros_np, out, idx_v, ones_v, hist_s):
        c = lax.axis_index("c")
        s = lax.axis_index("s")
        wid = c * NS + s

        @pl.when(s == 0)
        def _():
            pltpu.sync_copy(zeros_np, hist_s)

        for k in range(LANE // 16):
            ones_v[pl.ds(k * 16, 16)] = jnp.full((16,), 1.0, jnp.float32)
        plsc.subcore_barrier()

        base = wid * rows_per_w

        def chunk(i, carry):
            pltpu.sync_copy(dst2d.at[pl.ds(base + i * CROWS, CROWS)], idx_v)
            for j in range(CROWS):
                pltpu.sync_copy(ones_v, hist_s.at[idx_v.at[j]], add=True)
            return carry

        lax.fori_loop(0, n_chunks, chunk, 0)
        plsc.subcore_barrier()

        @pl.when(s == 0)
        def _():
            pltpu.sync_copy(hist_s, out.at[c])

    return hist_sc


def _make_edge_sc(np_pad, rows_per_w, n_chunks):
    mesh = plsc.VectorSubcoreMesh(core_axis_name="c", subcore_axis_name="s")

    @functools.partial(
        pl.kernel,
        out_type=jax.ShapeDtypeStruct((NC, np_pad), jnp.float32),
        mesh=mesh,
        scratch_types=[
            pltpu.VMEM((CROWS, LANE), jnp.int32),
            pltpu.VMEM((CROWS, LANE), jnp.int32),
            pltpu.VMEM((LANE,), jnp.float32),
            pltpu.VMEM_SHARED((np_pad,), jnp.float32),
            pltpu.VMEM_SHARED((np_pad,), jnp.float32),
        ],
    )
    def edge_sc(src2d, dst2d, t_hbm, zeros_np, out, sidx_v, didx_v, vals_v,
                t_s, acc_s):
        c = lax.axis_index("c")
        s = lax.axis_index("s")
        wid = c * NS + s

        @pl.when(s == 0)
        def _():
            pltpu.sync_copy(zeros_np, acc_s)

        @pl.when(s == 1)
        def _():
            pltpu.sync_copy(t_hbm, t_s)

        plsc.subcore_barrier()
        base = wid * rows_per_w

        def chunk(i, carry):
            row0 = base + i * CROWS
            pltpu.sync_copy(src2d.at[pl.ds(row0, CROWS)], sidx_v)
            pltpu.sync_copy(dst2d.at[pl.ds(row0, CROWS)], didx_v)
            for j in range(CROWS):
                pltpu.sync_copy(t_s.at[sidx_v.at[j]], vals_v)
                pltpu.sync_copy(vals_v, acc_s.at[didx_v.at[j]], add=True)
            return carry

        lax.fori_loop(0, n_chunks, chunk, 0)
        plsc.subcore_barrier()

        @pl.when(s == 0)
        def _():
            pltpu.sync_copy(acc_s, out.at[c])

    return edge_sc


def kernel(x, edge_index, face_feats, W_e1, b_e1, bn_gamma, bn_beta, prelu_a,
           W_e2, b_e2, W0, b0, Wg, bg, Wface, Wf, bf):
    n = x.shape[0]
    e = edge_index.shape[1]
    ff = face_feats.shape[1]
    np_pad = ((n + 1 + LANE - 1) // LANE) * LANE
    rows_raw = -(-e // (NW * LANE))
    rows_per_w = ((rows_raw + CROWS - 1) // CROWS) * CROWS
    n_chunks = rows_per_w // CROWS
    e_pad = NW * LANE * rows_per_w
    rows2 = np_pad // LANE

    bn = 2000
    n_blocks = n // bn

    f32 = jnp.float32
    b_e1r = b_e1.reshape(1, -1)
    gam = bn_gamma.reshape(1, -1)
    bet = bn_beta.reshape(1, -1)
    b_e2r = b_e2.reshape(1, -1)
    bgr = bg.reshape(1, -1)
    wft = Wf.reshape(1, -1)
    bb = jnp.concatenate([bf, b0]).astype(f32)

    stats = pl.pallas_call(
        _stats_body,
        out_shape=jax.ShapeDtypeStruct((8,), f32),
        out_specs=pl.BlockSpec(memory_space=pltpu.SMEM),
    )(x.T)

    wspec = lambda shp: pl.BlockSpec(shp, lambda i: (0, 0))
    s_col, lin_col = pl.pallas_call(
        functools.partial(_main_body, n),
        grid=(n_blocks,),
        in_specs=[
            pl.BlockSpec(memory_space=pltpu.SMEM),
            pl.BlockSpec(memory_space=pltpu.SMEM),
            pl.BlockSpec((bn, 2), lambda i: (i, 0)),
            pl.BlockSpec((bn, ff), lambda i: (i, 0)),
            wspec((2, 32)), wspec((1, 32)), wspec((1, 32)), wspec((1, 32)),
            wspec((32, 32)), wspec((1, 32)), wspec((32, 1)), wspec((32, 32)),
            wspec((ff, 32)), wspec((32, 1)),
        ],
        out_specs=[
            pl.BlockSpec((bn, 1), lambda i: (i, 0)),
            pl.BlockSpec((bn, 1), lambda i: (i, 0)),
        ],
        out_shape=[
            jax.ShapeDtypeStruct((n, 1), f32),
            jax.ShapeDtypeStruct((n, 1), f32),
        ],
    )(stats, prelu_a, x, face_feats, W_e1, b_e1r, gam, bet, W_e2, b_e2r,
      W0, Wg, Wface, Wf)

    src = edge_index[0]
    dst = edge_index[1]
    pad_e = e_pad - e
    src_p = jnp.concatenate([src, jnp.zeros((pad_e,), jnp.int32)])
    dst_p = jnp.concatenate([dst, jnp.full((pad_e,), n, jnp.int32)])
    src2d = src_p.reshape(-1, LANE)
    dst2d = dst_p.reshape(-1, LANE)
    zeros_np = jnp.zeros((np_pad,), f32)

    hist = _make_hist_sc(np_pad, rows_per_w, n_chunks)(dst2d, zeros_np)

    pad_n = np_pad - n
    s2 = jnp.pad(s_col[:, 0], (0, pad_n)).reshape(rows2, LANE)
    lin2 = jnp.pad(lin_col[:, 0], (0, pad_n)).reshape(rows2, LANE)
    dinv2, t2 = pl.pallas_call(
        _post_body,
        out_shape=[
            jax.ShapeDtypeStruct((rows2, LANE), f32),
            jax.ShapeDtypeStruct((rows2, LANE), f32),
        ],
    )(hist.reshape(NC, rows2, LANE), s2)

    acc = _make_edge_sc(np_pad, rows_per_w, n_chunks)(
        src2d, dst2d, t2.reshape(np_pad), zeros_np)

    out2 = pl.pallas_call(
        _final_body,
        out_shape=jax.ShapeDtypeStruct((rows2, LANE), f32),
        in_specs=[
            pl.BlockSpec((rows2, LANE), lambda: (0, 0)),
            pl.BlockSpec((rows2, LANE), lambda: (0, 0)),
            pl.BlockSpec((rows2, LANE), lambda: (0, 0)),
            pl.BlockSpec((NC, rows2, LANE), lambda: (0, 0, 0)),
            pl.BlockSpec((1, 32), lambda: (0, 0)),
            pl.BlockSpec((1, 32), lambda: (0, 0)),
            pl.BlockSpec(memory_space=pltpu.SMEM),
        ],
    )(lin2, s2, dinv2, acc.reshape(NC, rows2, LANE), bgr, wft, bb)

    return out2.reshape(np_pad)[:n]


# speedup vs baseline: 121.2256x; 1.4868x over previous
import functools

import jax
import jax.numpy as jnp
from jax import lax
from jax.experimental import pallas as pl
from jax.experimental.pallas import tpu as pltpu
from jax.experimental.pallas import tpu_sc as plsc

NC = 2
NS = 16
NW = NC * NS
LANE = 128
CR = 130


def _stats_body(x_ref, o_ref):
    i = pl.program_id(0)

    @pl.when(i == 0)
    def _():
        for k in range(5):
            o_ref[k] = 0.0

    xb = x_ref[...]
    x0 = xb[:, 0:1]
    x1 = xb[:, 1:2]
    o_ref[0] += jnp.sum(x0)
    o_ref[1] += jnp.sum(x1)
    o_ref[2] += jnp.sum(x0 * x0)
    o_ref[3] += jnp.sum(x0 * x1)
    o_ref[4] += jnp.sum(x1 * x1)


def _main_body(n_rows, stats_ref, pa_ref, x_ref, face_ref, we1_ref, be1_ref,
               gam_ref, bet_ref, we2_ref, be2_ref, w0_ref, wg_ref, wface_ref,
               wf_ref, s_out, lin_out):
    ninv = 1.0 / float(n_rows)
    s0 = stats_ref[0] * ninv
    s1 = stats_ref[1] * ninv
    c00 = stats_ref[2] * ninv - s0 * s0
    c01 = stats_ref[3] * ninv - s0 * s1
    c11 = stats_ref[4] * ninv - s1 * s1
    w0r = we1_ref[0:1, :]
    w1r = we1_ref[1:2, :]
    mu = s0 * w0r + s1 * w1r + be1_ref[...]
    var = w0r * w0r * c00 + 2.0 * w0r * w1r * c01 + w1r * w1r * c11
    inv = lax.rsqrt(var + 1e-5)
    h = jnp.dot(x_ref[...], we1_ref[...],
                preferred_element_type=jnp.float32) + be1_ref[...]
    hn = (h - mu) * (inv * gam_ref[...]) + bet_ref[...]
    a = pa_ref[0]
    hp = jnp.where(hn >= 0, hn, a * hn)
    e = jnp.dot(hp, we2_ref[...],
                preferred_element_type=jnp.float32) + be2_ref[...]
    lin_out[...] = jnp.dot(e, w0_ref[...], preferred_element_type=jnp.float32)
    m = jnp.dot(e, wg_ref[...], preferred_element_type=jnp.float32)
    m = m + jnp.dot(face_ref[...], wface_ref[...],
                    preferred_element_type=jnp.float32)
    s_out[...] = jnp.dot(m, wf_ref[...], preferred_element_type=jnp.float32)


def _post_body(hist_ref, s_ref, dinv_ref, t_ref):
    deg = hist_ref[0] + hist_ref[1] + 1.0
    dinv = lax.rsqrt(deg)
    dinv_ref[...] = dinv
    t_ref[...] = s_ref[...] * dinv


def _final_body(lin_ref, s_ref, dinv_ref, acc_ref, bg_ref, wft_ref, bb_ref,
                out_ref):
    const = jnp.sum(bg_ref[...] * wft_ref[...]) + bb_ref[0] + bb_ref[1]
    dinv = dinv_ref[...]
    out_ref[...] = (lin_ref[...] + (acc_ref[0] + acc_ref[1]) * dinv
                    + s_ref[...] * dinv * dinv + const)


def _make_hist_sc(np_pad, rows_main, n_chunks, tail_rows):
    mesh = plsc.VectorSubcoreMesh(core_axis_name="c", subcore_axis_name="s")

    @functools.partial(
        pl.kernel,
        out_type=jax.ShapeDtypeStruct((NC, np_pad), jnp.float32),
        mesh=mesh,
        scratch_types=[
            pltpu.VMEM((CR * LANE,), jnp.int32),
            pltpu.VMEM((CR * LANE,), jnp.float32),
            pltpu.VMEM((LANE,), jnp.int32),
            pltpu.VMEM((LANE,), jnp.float32),
            pltpu.VMEM_SHARED((np_pad,), jnp.float32),
        ],
    )
    def hist_sc(ei, ones_hbm, zeros_np, out, idx_v, ones_v, tidx_v, tones_v,
                hist_s):
        c = lax.axis_index("c")
        s = lax.axis_index("s")
        wid = c * NS + s

        @pl.when(s == 0)
        def _():
            pltpu.sync_copy(zeros_np, hist_s)

        pltpu.sync_copy(ones_hbm, ones_v)
        pltpu.sync_copy(ones_hbm.at[pl.ds(0, LANE)], tones_v)
        plsc.subcore_barrier()

        base = wid * rows_main * LANE

        def chunk(i, carry):
            off = base + i * (CR * LANE)
            pltpu.sync_copy(ei.at[1, pl.ds(off, CR * LANE)], idx_v)
            pltpu.sync_copy(ones_v, hist_s.at[idx_v], add=True)
            return carry

        lax.fori_loop(0, n_chunks, chunk, 0)

        @pl.when(wid < tail_rows)
        def _():
            toff = (NW * rows_main + wid) * LANE
            pltpu.sync_copy(ei.at[1, pl.ds(toff, LANE)], tidx_v)
            pltpu.sync_copy(tones_v, hist_s.at[tidx_v], add=True)

        plsc.subcore_barrier()

        @pl.when(s == 0)
        def _():
            pltpu.sync_copy(hist_s, out.at[c])

    return hist_sc


def _make_edge_sc(np_pad, rows_main, n_chunks, tail_rows):
    mesh = plsc.VectorSubcoreMesh(core_axis_name="c", subcore_axis_name="s")

    @functools.partial(
        pl.kernel,
        out_type=jax.ShapeDtypeStruct((NC, np_pad), jnp.float32),
        mesh=mesh,
        scratch_types=[
            pltpu.VMEM((CR * LANE,), jnp.int32),
            pltpu.VMEM((CR * LANE,), jnp.int32),
            pltpu.VMEM((CR * LANE,), jnp.float32),
            pltpu.VMEM((LANE,), jnp.int32),
            pltpu.VMEM((LANE,), jnp.int32),
            pltpu.VMEM((LANE,), jnp.float32),
            pltpu.VMEM_SHARED((np_pad,), jnp.float32),
            pltpu.VMEM_SHARED((np_pad,), jnp.float32),
        ],
    )
    def edge_sc(ei, t_hbm, zeros_np, out, sidx_v, didx_v, vals_v, tsidx_v,
                tdidx_v, tvals_v, t_s, acc_s):
        c = lax.axis_index("c")
        s = lax.axis_index("s")
        wid = c * NS + s

        @pl.when(s == 0)
        def _():
            pltpu.sync_copy(zeros_np, acc_s)

        @pl.when(s == 1)
        def _():
            pltpu.sync_copy(t_hbm, t_s)

        plsc.subcore_barrier()
        base = wid * rows_main * LANE

        def chunk(i, carry):
            off = base + i * (CR * LANE)
            pltpu.sync_copy(ei.at[0, pl.ds(off, CR * LANE)], sidx_v)
            pltpu.sync_copy(ei.at[1, pl.ds(off, CR * LANE)], didx_v)
            pltpu.sync_copy(t_s.at[sidx_v], vals_v)
            pltpu.sync_copy(vals_v, acc_s.at[didx_v], add=True)
            return carry

        lax.fori_loop(0, n_chunks, chunk, 0)

        @pl.when(wid < tail_rows)
        def _():
            toff = (NW * rows_main + wid) * LANE
            pltpu.sync_copy(ei.at[0, pl.ds(toff, LANE)], tsidx_v)
            pltpu.sync_copy(ei.at[1, pl.ds(toff, LANE)], tdidx_v)
            pltpu.sync_copy(t_s.at[tsidx_v], tvals_v)
            pltpu.sync_copy(tvals_v, acc_s.at[tdidx_v], add=True)

        plsc.subcore_barrier()

        @pl.when(s == 0)
        def _():
            pltpu.sync_copy(acc_s, out.at[c])

    return edge_sc


def kernel(x, edge_index, face_feats, W_e1, b_e1, bn_gamma, bn_beta, prelu_a,
           W_e2, b_e2, W0, b0, Wg, bg, Wface, Wf, bf):
    n = x.shape[0]
    e = edge_index.shape[1]
    ff = face_feats.shape[1]
    np_pad = ((n + LANE - 1) // LANE) * LANE
    rows2 = np_pad // LANE

    e_rows = e // LANE
    rows_main = (e_rows // NW // CR) * CR
    n_chunks = rows_main // CR
    tail_rows = e_rows - NW * rows_main
    assert rows_main * NW + tail_rows == e_rows and tail_rows <= NW

    bn = 2000
    n_blocks = n // bn
    sb = 10000
    s_blocks = n // sb

    f32 = jnp.float32
    b_e1r = b_e1.reshape(1, -1)
    gam = bn_gamma.reshape(1, -1)
    bet = bn_beta.reshape(1, -1)
    b_e2r = b_e2.reshape(1, -1)
    bgr = bg.reshape(1, -1)
    wft = Wf.reshape(1, -1)
    bb = jnp.concatenate([bf, b0]).astype(f32)
    zeros_np = jnp.zeros((np_pad,), f32)
    ones_cr = jnp.ones((CR * LANE,), f32)

    stats = pl.pallas_call(
        _stats_body,
        grid=(s_blocks,),
        in_specs=[pl.BlockSpec((sb, 2), lambda i: (i, 0))],
        out_specs=pl.BlockSpec(memory_space=pltpu.SMEM),
        out_shape=jax.ShapeDtypeStruct((8,), f32),
    )(x)

    hist = _make_hist_sc(np_pad, rows_main, n_chunks, tail_rows)(
        edge_index, ones_cr, zeros_np)

    wspec = lambda shp: pl.BlockSpec(shp, lambda i: (0, 0))
    s_col, lin_col = pl.pallas_call(
        functools.partial(_main_body, n),
        grid=(n_blocks,),
        in_specs=[
            pl.BlockSpec(memory_space=pltpu.SMEM),
            pl.BlockSpec(memory_space=pltpu.SMEM),
            pl.BlockSpec((bn, 2), lambda i: (i, 0)),
            pl.BlockSpec((bn, ff), lambda i: (i, 0)),
            wspec((2, 32)), wspec((1, 32)), wspec((1, 32)), wspec((1, 32)),
            wspec((32, 32)), wspec((1, 32)), wspec((32, 1)), wspec((32, 32)),
            wspec((ff, 32)), wspec((32, 1)),
        ],
        out_specs=[
            pl.BlockSpec((bn, 1), lambda i: (i, 0)),
            pl.BlockSpec((bn, 1), lambda i: (i, 0)),
        ],
        out_shape=[
            jax.ShapeDtypeStruct((n, 1), f32),
            jax.ShapeDtypeStruct((n, 1), f32),
        ],
    )(stats, prelu_a, x, face_feats, W_e1, b_e1r, gam, bet, W_e2, b_e2r,
      W0, Wg, Wface, Wf)

    pad_n = np_pad - n
    s2 = jnp.pad(s_col[:, 0], (0, pad_n)).reshape(rows2, LANE)
    lin2 = jnp.pad(lin_col[:, 0], (0, pad_n)).reshape(rows2, LANE)
    dinv2, t2 = pl.pallas_call(
        _post_body,
        out_shape=[
            jax.ShapeDtypeStruct((rows2, LANE), f32),
            jax.ShapeDtypeStruct((rows2, LANE), f32),
        ],
    )(hist.reshape(NC, rows2, LANE), s2)

    acc = _make_edge_sc(np_pad, rows_main, n_chunks, tail_rows)(
        edge_index, t2.reshape(np_pad), zeros_np)

    out2 = pl.pallas_call(
        _final_body,
        out_shape=jax.ShapeDtypeStruct((rows2, LANE), f32),
        in_specs=[
            pl.BlockSpec((rows2, LANE), lambda: (0, 0)),
            pl.BlockSpec((rows2, LANE), lambda: (0, 0)),
            pl.BlockSpec((rows2, LANE), lambda: (0, 0)),
            pl.BlockSpec((NC, rows2, LANE), lambda: (0, 0, 0)),
            pl.BlockSpec((1, 32), lambda: (0, 0)),
            pl.BlockSpec((1, 32), lambda: (0, 0)),
            pl.BlockSpec(memory_space=pltpu.SMEM),
        ],
    )(lin2, s2, dinv2, acc.reshape(NC, rows2, LANE), bgr, wft, bb)

    return out2.reshape(np_pad)[:n]
